# Initial kernel scaffold; baseline (speedup 1.0000x reference)
import jax
import jax.numpy as jnp
from jax.experimental import pallas as pl


def kernel(x, qr, cos, sin, Wq_b, Wk, Wwp, ln_w, ln_b):
    raise NotImplementedError("write your pallas kernel here")


# speedup vs baseline: 2.8275x; 2.8275x over previous
import jax
import jax.numpy as jnp
from jax.experimental import pallas as pl

DIM = 7168
NH = 64
HD = 128
RD = 64
QLR = 1536
TOPK = 2048
T = 2048

TB = 128
HC = 8
QNB = 1024


def _rope_rot(q, ce, se):
    lane = jax.lax.broadcasted_iota(jnp.int32, q.shape, 1)
    even = (lane % 2) == 0
    swapped = jnp.where(even, jnp.roll(q, -1, axis=-1), jnp.roll(q, 1, axis=-1))
    return q * ce + swapped * se


def _q_body(qr_ref, wqb_ref, ce_ref, se_ref, out_ref):
    q = jnp.dot(qr_ref[...], wqb_ref[...], preferred_element_type=jnp.float32)
    ce = ce_ref[...]
    se = se_ref[...]
    for j in range(QNB // HD):
        out_ref[:, j * HD:(j + 1) * HD] = _rope_rot(
            q[:, j * HD:(j + 1) * HD], ce, se)


def _q_proj(qr_bf, wqb_bf, ce_h, se_h):
    grid = (NH * HD // QNB,)
    return pl.pallas_call(
        _q_body,
        grid=grid,
        in_specs=[
            pl.BlockSpec((T, QLR), lambda n: (0, 0)),
            pl.BlockSpec((QLR, QNB), lambda n: (0, n)),
            pl.BlockSpec((T, HD), lambda n: (0, 0)),
            pl.BlockSpec((T, HD), lambda n: (0, 0)),
        ],
        out_specs=pl.BlockSpec((T, QNB), lambda n: (0, n)),
        out_shape=jax.ShapeDtypeStruct((T, NH * HD), jnp.float32),
    )(qr_bf, wqb_bf, ce_h, se_h)


def _kw_body(x_ref, wk_ref, wwp_ref, ce1_ref, se1_ref, lnw_ref, lnb_ref,
             k_ref, w_ref):
    xb = x_ref[...]
    kw = jnp.dot(xb, wk_ref[...], preferred_element_type=jnp.float32)
    mu = jnp.mean(kw, axis=-1, keepdims=True)
    var = jnp.var(kw, axis=-1, keepdims=True)
    ln = (kw - mu) / jnp.sqrt(var + 1e-6) * lnw_ref[...] + lnb_ref[...]
    k_ref[...] = _rope_rot(ln, ce1_ref[...], se1_ref[...])
    w_ref[...] = jnp.dot(xb, wwp_ref[...], preferred_element_type=jnp.float32)


def _kw_proj(x_bf, wk_bf, wwp_bf, ce1, se1, ln_w, ln_b):
    return pl.pallas_call(
        _kw_body,
        out_shape=(jax.ShapeDtypeStruct((T, HD), jnp.float32),
                   jax.ShapeDtypeStruct((T, NH), jnp.float32)),
    )(x_bf, wk_bf, wwp_bf, ce1, se1, ln_w.reshape(1, HD), ln_b.reshape(1, HD))


def _scores_body(q_ref, kt_ref, w_ref, out_ref):
    hc = pl.program_id(1)
    tb = pl.program_id(0)
    rot = q_ref[...]

    @pl.when(hc == 0)
    def _():
        out_ref[...] = jnp.zeros_like(out_ref)

    w = w_ref[0]
    for j in range(HC):
        qh = rot[:, j * HD:(j + 1) * HD]
        s = jnp.dot(qh.astype(jnp.bfloat16), kt_ref[...],
                    preferred_element_type=jnp.float32)
        out_ref[...] += w[:, j:j + 1] * jnp.maximum(s, 0.0)

    @pl.when(hc == HC - 1)
    def _():
        scale = HD ** (-0.5)
        row = tb * TB + jax.lax.broadcasted_iota(jnp.int32, out_ref.shape, 0)
        col = jax.lax.broadcasted_iota(jnp.int32, out_ref.shape, 1)
        out_ref[...] = jnp.where(col > row, -1e9, out_ref[...] * scale)


def _compute_scores(q_rot, kt_bf, w_r):
    grid = (T // TB, NH // HC)
    return pl.pallas_call(
        _scores_body,
        grid=grid,
        in_specs=[
            pl.BlockSpec((TB, HC * HD), lambda t, h: (t, h)),
            pl.BlockSpec((HD, T), lambda t, h: (0, 0)),
            pl.BlockSpec((1, TB, HC), lambda t, h: (h, t, 0)),
        ],
        out_specs=pl.BlockSpec((TB, T), lambda t, h: (t, 0)),
        out_shape=jax.ShapeDtypeStruct((T, T), jnp.float32),
    )(q_rot, kt_bf, w_r)


def _rope_coeffs(cos, sin, nh):
    ce_h = jnp.repeat(cos, 2, axis=1)
    sgn = jnp.where(jnp.arange(RD) % 2 == 0, -1.0, 1.0)
    se_h = jnp.repeat(sin, 2, axis=1) * sgn[None, :]
    ce_h = jnp.concatenate([ce_h, jnp.ones((T, HD - RD), jnp.float32)], axis=1)
    se_h = jnp.concatenate([se_h, jnp.zeros((T, HD - RD), jnp.float32)], axis=1)
    if nh == 1:
        return ce_h, se_h
    return jnp.tile(ce_h, (1, nh)), jnp.tile(se_h, (1, nh))


def kernel(x, qr, cos, sin, Wq_b, Wk, Wwp, ln_w, ln_b):
    ce1, se1 = _rope_coeffs(cos, sin, 1)

    k, w = _kw_proj(x.astype(jnp.bfloat16), Wk.astype(jnp.bfloat16),
                    Wwp.astype(jnp.bfloat16), ce1, se1, ln_w, ln_b)
    q_rot = _q_proj(qr.astype(jnp.bfloat16), Wq_b.astype(jnp.bfloat16), ce1, se1)

    kt_bf = k.T.astype(jnp.bfloat16)
    w_r = jnp.transpose(w.reshape(T, NH // HC, HC), (1, 0, 2))
    scores = _compute_scores(q_rot, kt_bf, w_r)
    vals, idx = jax.lax.top_k(scores, TOPK)
    return vals, idx.astype(jnp.int32)
